# baseline (speedup 1.0000x reference)
import functools

import jax
import jax.numpy as jnp
from jax.experimental import pallas as pl
from jax.experimental.pallas import tpu as pltpu

NUM_QUANTIZERS = 8
CODEBOOK_SIZE = 8192
DIM = 32
B = 32
N = 1024
TOKENS = B * N
BLOCK_T = 256


def _rvq_kernel(x_ref, cb_ref, cbt_ref, qout_ref, idx_ref, commit_ref):
    T = x_ref.shape[0]
    K = cb_ref.shape[0]
    x0 = x_ref[...]
    cb = cb_ref[...]
    cbt = cbt_ref[...]
    cb_sq = jnp.sum(cbt * cbt, axis=0, keepdims=True)
    iota = jax.lax.broadcasted_iota(jnp.int32, (T, K), 1)
    lane_q = jax.lax.broadcasted_iota(jnp.int32, (1, NUM_QUANTIZERS), 1)

    @pl.when(pl.program_id(0) == 0)
    def _():
        commit_ref[...] = jnp.zeros_like(commit_ref)

    inv_count = 1.0 / (TOKENS * DIM)
    r = x0
    idx_cols = []
    commit_acc = jnp.zeros((1, NUM_QUANTIZERS), jnp.float32)
    for q in range(NUM_QUANTIZERS):
        prod = jax.lax.dot_general(
            r, cbt, (((1,), (0,)), ((), ())),
            precision=jax.lax.Precision.DEFAULT,
            preferred_element_type=jnp.float32,
        )
        r_sq = jnp.sum(r * r, axis=1, keepdims=True)
        s = (r_sq - 2.0 * prod) + cb_sq
        m = jnp.min(s, axis=1, keepdims=True)
        idx = jnp.min(jnp.where(s == m, iota, K), axis=1, keepdims=True)
        onehot = (iota == idx).astype(jnp.float32)
        quant = jax.lax.dot_general(
            onehot, cb, (((1,), (0,)), ((), ())),
            precision=jax.lax.Precision.HIGHEST,
            preferred_element_type=jnp.float32,
        )
        diff = quant - r
        part = jnp.sum(diff * diff) * inv_count
        commit_acc = commit_acc + jnp.where(lane_q == q, part, 0.0)
        r = r - quant
        idx_cols.append(idx)

    qout_ref[...] = x0 - r
    idx_ref[...] = jnp.concatenate(idx_cols, axis=1)
    commit_ref[...] = commit_ref[...] + commit_acc


@jax.jit
def kernel(x, codebook):
    xf = x.reshape(TOKENS, DIM)
    cbt = codebook.T
    grid = (TOKENS // BLOCK_T,)
    qout, idx, commit = pl.pallas_call(
        _rvq_kernel,
        grid=grid,
        in_specs=[
            pl.BlockSpec((BLOCK_T, DIM), lambda i: (i, 0)),
            pl.BlockSpec((CODEBOOK_SIZE, DIM), lambda i: (0, 0)),
            pl.BlockSpec((DIM, CODEBOOK_SIZE), lambda i: (0, 0)),
        ],
        out_specs=[
            pl.BlockSpec((BLOCK_T, DIM), lambda i: (i, 0)),
            pl.BlockSpec((BLOCK_T, NUM_QUANTIZERS), lambda i: (i, 0)),
            pl.BlockSpec((1, NUM_QUANTIZERS), lambda i: (0, 0)),
        ],
        out_shape=[
            jax.ShapeDtypeStruct((TOKENS, DIM), jnp.float32),
            jax.ShapeDtypeStruct((TOKENS, NUM_QUANTIZERS), jnp.int32),
            jax.ShapeDtypeStruct((1, NUM_QUANTIZERS), jnp.float32),
        ],
    )(xf, codebook, cbt)
    return (
        qout.reshape(B, N, DIM),
        idx.reshape(B, N, NUM_QUANTIZERS),
        commit.reshape(NUM_QUANTIZERS),
    )


# speedup vs baseline: 1.0026x; 1.0026x over previous
import jax
import jax.numpy as jnp
from jax.experimental import pallas as pl
from jax.experimental.pallas import tpu as pltpu

NUM_QUANTIZERS = 8
CODEBOOK_SIZE = 8192
DIM = 32
B = 32
N = 1024
TOKENS = B * N
BLOCK_T = 256
GRID = TOKENS // BLOCK_T


def _rvq_kernel(x_ref, cbt_ref, cbb_ref, qout_ref, idx_ref, commit_ref):
    T = x_ref.shape[0]
    K = cbt_ref.shape[1]
    x0 = x_ref[...]
    cbt = cbt_ref[...]
    cbb = cbb_ref[...]
    cb_sq = jnp.sum(cbt * cbt, axis=0, keepdims=True)
    iota = jax.lax.broadcasted_iota(jnp.int32, (T, K), 1)

    inv_count = 1.0 / (TOKENS * DIM)
    r = x0
    idx_cols = []
    commit_cols = []
    for q in range(NUM_QUANTIZERS):
        prod = jax.lax.dot_general(
            -2.0 * r, cbt, (((1,), (0,)), ((), ())),
            precision=jax.lax.Precision.DEFAULT,
            preferred_element_type=jnp.float32,
        )
        r_sq = jnp.sum(r * r, axis=1, keepdims=True)
        s = (r_sq + prod) + cb_sq
        m = jnp.min(s, axis=1, keepdims=True)
        idx = jnp.min(jnp.where(s == m, iota, K), axis=1, keepdims=True)
        onehot = (iota == idx).astype(jnp.float32)
        quant = jax.lax.dot_general(
            onehot, cbb, (((1,), (0,)), ((), ())),
            precision=jax.lax.Precision.HIGHEST,
            preferred_element_type=jnp.float32,
        )
        r = r - quant
        commit_cols.append(jnp.sum(r * r) * inv_count)
        idx_cols.append(idx)

    qout_ref[...] = x0 - r
    idx_ref[...] = jnp.concatenate(idx_cols, axis=1)
    lane_q = jax.lax.broadcasted_iota(jnp.int32, (1, NUM_QUANTIZERS), 1)
    acc = jnp.zeros((1, NUM_QUANTIZERS), jnp.float32)
    for q in range(NUM_QUANTIZERS):
        acc = acc + jnp.where(lane_q == q, commit_cols[q], 0.0)
    commit_ref[0] = acc


def _encode_codebook_bytes(codebook):
    return codebook


@jax.jit
def kernel(x, codebook):
    xf = x.reshape(TOKENS, DIM)
    cbt = codebook.T
    cbb = _encode_codebook_bytes(codebook)
    qout, idx, commit = pl.pallas_call(
        _rvq_kernel,
        grid=(GRID,),
        in_specs=[
            pl.BlockSpec((BLOCK_T, DIM), lambda i: (i, 0)),
            pl.BlockSpec((DIM, CODEBOOK_SIZE), lambda i: (0, 0)),
            pl.BlockSpec((CODEBOOK_SIZE, DIM), lambda i: (0, 0)),
        ],
        out_specs=[
            pl.BlockSpec((BLOCK_T, DIM), lambda i: (i, 0)),
            pl.BlockSpec((BLOCK_T, NUM_QUANTIZERS), lambda i: (i, 0)),
            pl.BlockSpec((1, 1, NUM_QUANTIZERS), lambda i: (i, 0, 0)),
        ],
        out_shape=[
            jax.ShapeDtypeStruct((TOKENS, DIM), jnp.float32),
            jax.ShapeDtypeStruct((TOKENS, NUM_QUANTIZERS), jnp.int32),
            jax.ShapeDtypeStruct((GRID, 1, NUM_QUANTIZERS), jnp.float32),
        ],
        compiler_params=pltpu.CompilerParams(
            dimension_semantics=("parallel",),
        ),
    )(xf, cbt, cbb)
    return (
        qout.reshape(B, N, DIM),
        idx.reshape(B, N, NUM_QUANTIZERS),
        jnp.sum(commit, axis=0).reshape(NUM_QUANTIZERS),
    )


# speedup vs baseline: 2.4935x; 2.4869x over previous
import jax
import jax.numpy as jnp
from jax.experimental import pallas as pl
from jax.experimental.pallas import tpu as pltpu

NUM_QUANTIZERS = 8
CODEBOOK_SIZE = 8192
DIM = 32
B = 32
N = 1024
TOKENS = B * N
BLOCK_T = 256
GRID = TOKENS // BLOCK_T


def _rvq_kernel(x_ref, cbt_ref, cbb_ref, qout_ref, idx_ref, commit_ref):
    T = x_ref.shape[0]
    K = cbt_ref.shape[1]
    x0 = x_ref[...]
    cbt = cbt_ref[...]
    cbb = cbb_ref[...]
    cb_sq = jnp.sum(cbt * cbt, axis=0, keepdims=True)
    iota = jax.lax.broadcasted_iota(jnp.int32, (T, K), 1)

    inv_count = 1.0 / (TOKENS * DIM)
    r = x0
    idx_cols = []
    commit_cols = []
    for q in range(NUM_QUANTIZERS):
        prod = jax.lax.dot_general(
            -2.0 * r, cbt, (((1,), (0,)), ((), ())),
            precision=jax.lax.Precision.DEFAULT,
            preferred_element_type=jnp.float32,
        )
        r_sq = jnp.sum(r * r, axis=1, keepdims=True)
        s = (r_sq + prod) + cb_sq
        m = jnp.min(s, axis=1, keepdims=True)
        idx = jnp.min(jnp.where(s == m, iota, K), axis=1, keepdims=True)
        onehot = (iota == idx).astype(jnp.float32)
        gb = jax.lax.dot_general(
            onehot, cbb, (((1,), (0,)), ((), ())),
            precision=jax.lax.Precision.DEFAULT,
            preferred_element_type=jnp.float32,
        )
        quant = (gb[:, 0:DIM] + gb[:, DIM:2 * DIM]) + gb[:, 2 * DIM:3 * DIM]
        r = r - quant
        commit_cols.append(jnp.sum(r * r) * inv_count)
        idx_cols.append(idx)

    qout_ref[...] = x0 - r
    idx_ref[...] = jnp.concatenate(idx_cols, axis=1)
    lane_q = jax.lax.broadcasted_iota(jnp.int32, (1, NUM_QUANTIZERS), 1)
    acc = jnp.zeros((1, NUM_QUANTIZERS), jnp.float32)
    for q in range(NUM_QUANTIZERS):
        acc = acc + jnp.where(lane_q == q, commit_cols[q], 0.0)
    commit_ref[0] = acc


def _encode_codebook_bytes(codebook):
    hi = jnp.bfloat16(codebook).astype(jnp.float32)
    r1 = codebook - hi
    mid = jnp.bfloat16(r1).astype(jnp.float32)
    lo = jnp.bfloat16(r1 - mid).astype(jnp.float32)
    return jnp.concatenate([hi, mid, lo], axis=1)


@jax.jit
def kernel(x, codebook):
    xf = x.reshape(TOKENS, DIM)
    cbt = codebook.T
    cbb = _encode_codebook_bytes(codebook)
    qout, idx, commit = pl.pallas_call(
        _rvq_kernel,
        grid=(GRID,),
        in_specs=[
            pl.BlockSpec((BLOCK_T, DIM), lambda i: (i, 0)),
            pl.BlockSpec((DIM, CODEBOOK_SIZE), lambda i: (0, 0)),
            pl.BlockSpec((CODEBOOK_SIZE, 3 * DIM), lambda i: (0, 0)),
        ],
        out_specs=[
            pl.BlockSpec((BLOCK_T, DIM), lambda i: (i, 0)),
            pl.BlockSpec((BLOCK_T, NUM_QUANTIZERS), lambda i: (i, 0)),
            pl.BlockSpec((1, 1, NUM_QUANTIZERS), lambda i: (i, 0, 0)),
        ],
        out_shape=[
            jax.ShapeDtypeStruct((TOKENS, DIM), jnp.float32),
            jax.ShapeDtypeStruct((TOKENS, NUM_QUANTIZERS), jnp.int32),
            jax.ShapeDtypeStruct((GRID, 1, NUM_QUANTIZERS), jnp.float32),
        ],
        compiler_params=pltpu.CompilerParams(
            dimension_semantics=("parallel",),
        ),
    )(xf, cbt, cbb)
    return (
        qout.reshape(B, N, DIM),
        idx.reshape(B, N, NUM_QUANTIZERS),
        jnp.sum(commit, axis=0).reshape(NUM_QUANTIZERS),
    )
